# baseline (speedup 1.0000x reference)
import functools

import jax
import jax.numpy as jnp
from jax import lax
from jax.experimental import pallas as pl
from jax.experimental.pallas import tpu as pltpu
from jax.experimental.pallas import tpu_sc as plsc

B = 16384
D = 16
RAW = 3 * D
OUT = 64
NC = 2
NS = 16
NW = NC * NS
CHUNK = 128
PER_W = B // NW
CH = PER_W // CHUNK

_MESH = plsc.VectorSubcoreMesh(
    core_axis_name="c", subcore_axis_name="s", num_cores=NC, num_subcores=NS
)


@functools.partial(
    pl.kernel,
    out_type=[jax.ShapeDtypeStruct((NW * CH, CHUNK, D), jnp.float32)] * 3,
    mesh=_MESH,
    compiler_params=pltpu.CompilerParams(use_tc_tiling_on_sc=False),
    scratch_types=[
        pltpu.VMEM((CH, CHUNK), jnp.int32),
        pltpu.VMEM((CH, CHUNK), jnp.int32),
        pltpu.VMEM((CH, CHUNK), jnp.int32),
        pltpu.VMEM((CH, CHUNK, D), jnp.float32),
        pltpu.VMEM((CH, CHUNK, D), jnp.float32),
        pltpu.VMEM((CH, CHUNK, D), jnp.float32),
        pltpu.SemaphoreType.DMA,
    ],
)
def _gather3(i1, i2, i3, t1, t2, t3, o1, o2, o3, v1, v2, v3, r1, r2, r3, sem):
    wid = lax.axis_index("s") * NC + lax.axis_index("c")
    base = wid * CH
    pltpu.sync_copy(i1.at[pl.ds(base, CH)], v1)
    pltpu.sync_copy(i2.at[pl.ds(base, CH)], v2)
    pltpu.sync_copy(i3.at[pl.ds(base, CH)], v3)
    copies = []
    for tab, v, r in ((t1, v1, r1), (t2, v2, r2), (t3, v3, r3)):
        for j in range(CH):
            copies.append(pltpu.async_copy(tab.at[v.at[j]], r.at[j], sem))
    for c in copies:
        c.wait()
    pltpu.sync_copy(r1, o1.at[pl.ds(base, CH)])
    pltpu.sync_copy(r2, o2.at[pl.ds(base, CH)])
    pltpu.sync_copy(r3, o3.at[pl.ds(base, CH)])


def _proj_body(e1_ref, e2_ref, e3_ref, w_ref, b_ref, o_ref):
    c = jnp.concatenate([e1_ref[...], e2_ref[...], e3_ref[...]], axis=-1)
    acc = jnp.dot(c, w_ref[...], preferred_element_type=jnp.float32)
    o_ref[...] = jnp.maximum(acc + b_ref[0, :], 0.0)


_BLK = 4096


def _project(e1, e2, e3, W, b2d):
    return pl.pallas_call(
        _proj_body,
        grid=(B // _BLK,),
        in_specs=[
            pl.BlockSpec((_BLK, D), lambda i: (i, 0)),
            pl.BlockSpec((_BLK, D), lambda i: (i, 0)),
            pl.BlockSpec((_BLK, D), lambda i: (i, 0)),
            pl.BlockSpec((RAW, OUT), lambda i: (0, 0)),
            pl.BlockSpec((8, OUT), lambda i: (0, 0)),
        ],
        out_specs=pl.BlockSpec((_BLK, OUT), lambda i: (i, 0)),
        out_shape=jax.ShapeDtypeStruct((B, OUT), jnp.float32),
    )(e1, e2, e3, W, b2d)


def kernel(category_l1, category_l2, category_l3, E1, E2, E3, W, b):
    i1 = category_l1.astype(jnp.int32).reshape(NW * CH, CHUNK)
    i2 = category_l2.astype(jnp.int32).reshape(NW * CH, CHUNK)
    i3 = category_l3.astype(jnp.int32).reshape(NW * CH, CHUNK)
    g1, g2, g3 = _gather3(i1, i2, i3, E1, E2, E3)
    e1 = g1.reshape(B, D)
    e2 = g2.reshape(B, D)
    e3 = g3.reshape(B, D)
    b2d = jnp.broadcast_to(b, (8, OUT))
    return _project(e1, e2, e3, W, b2d)


# speedup vs baseline: 1.0318x; 1.0318x over previous
import functools

import jax
import jax.numpy as jnp
from jax import lax
from jax.experimental import pallas as pl
from jax.experimental.pallas import tpu as pltpu
from jax.experimental.pallas import tpu_sc as plsc

B = 16384
D = 16
RAW = 3 * D
OUT = 64
NC = 2
NS = 16
NW = NC * NS
CHUNK = 128
PER_W = B // NW
CH = PER_W // CHUNK

_MESH = plsc.VectorSubcoreMesh(
    core_axis_name="c", subcore_axis_name="s", num_cores=NC, num_subcores=NS
)


@functools.partial(
    pl.kernel,
    out_type=[jax.ShapeDtypeStruct((NW * CH, CHUNK, D), jnp.float32)] * 3,
    mesh=_MESH,
    compiler_params=pltpu.CompilerParams(use_tc_tiling_on_sc=False),
    scratch_types=[
        pltpu.VMEM((PER_W,), jnp.int32),
        pltpu.VMEM((PER_W,), jnp.int32),
        pltpu.VMEM((PER_W,), jnp.int32),
        pltpu.VMEM((CH, CHUNK, D), jnp.float32),
        pltpu.VMEM((CH, CHUNK, D), jnp.float32),
        pltpu.VMEM((CH, CHUNK, D), jnp.float32),
        pltpu.SemaphoreType.DMA,
    ],
)
def _gather3(i1, i2, i3, t1, t2, t3, o1, o2, o3, v1, v2, v3, r1, r2, r3, sem):
    wid = lax.axis_index("s") * NC + lax.axis_index("c")
    base = wid * PER_W
    pltpu.sync_copy(i1.at[pl.ds(base, PER_W)], v1)
    pltpu.sync_copy(i2.at[pl.ds(base, PER_W)], v2)
    pltpu.sync_copy(i3.at[pl.ds(base, PER_W)], v3)
    copies = []
    for tab, v, r in ((t1, v1, r1), (t2, v2, r2), (t3, v3, r3)):
        for j in range(CH):
            copies.append(
                pltpu.async_copy(tab.at[v.at[pl.ds(j * CHUNK, CHUNK)]], r.at[j], sem)
            )
    for c in copies:
        c.wait()
    pltpu.sync_copy(r1, o1.at[pl.ds(wid * CH, CH)])
    pltpu.sync_copy(r2, o2.at[pl.ds(wid * CH, CH)])
    pltpu.sync_copy(r3, o3.at[pl.ds(wid * CH, CH)])


_BLK_P = 256
_BLK_B = _BLK_P * 8


def _proj_body(p1_ref, p2_ref, p3_ref, m_ref, b_ref, o_ref):
    bias = b_ref[0, :]
    for j in range(8):
        acc = jnp.dot(p1_ref[...], m_ref[0, j], preferred_element_type=jnp.float32)
        acc += jnp.dot(p2_ref[...], m_ref[1, j], preferred_element_type=jnp.float32)
        acc += jnp.dot(p3_ref[...], m_ref[2, j], preferred_element_type=jnp.float32)
        o_ref[:, j, :] = jnp.maximum(acc + bias, 0.0)


def _project(p1, p2, p3, M, b2d):
    return pl.pallas_call(
        _proj_body,
        grid=(B // _BLK_B,),
        in_specs=[
            pl.BlockSpec((_BLK_P, 128), lambda i: (i, 0)),
            pl.BlockSpec((_BLK_P, 128), lambda i: (i, 0)),
            pl.BlockSpec((_BLK_P, 128), lambda i: (i, 0)),
            pl.BlockSpec((3, 8, 128, OUT), lambda i: (0, 0, 0, 0)),
            pl.BlockSpec((8, OUT), lambda i: (0, 0)),
        ],
        out_specs=pl.BlockSpec((_BLK_P, 8, OUT), lambda i: (i, 0, 0)),
        out_shape=jax.ShapeDtypeStruct((B // 8, 8, OUT), jnp.float32),
    )(p1, p2, p3, M, b2d)


def kernel(category_l1, category_l2, category_l3, E1, E2, E3, W, b):
    i1 = category_l1.astype(jnp.int32)
    i2 = category_l2.astype(jnp.int32)
    i3 = category_l3.astype(jnp.int32)
    g1, g2, g3 = _gather3(i1, i2, i3, E1, E2, E3)
    p1 = g1.reshape(B * D // 128, 128)
    p2 = g2.reshape(B * D // 128, 128)
    p3 = g3.reshape(B * D // 128, 128)
    sel = (
        jnp.arange(128)[None, :, None]
        == 16 * jnp.arange(8)[:, None, None] + jnp.arange(D)[None, None, :]
    ).astype(jnp.float32)
    w3 = W.reshape(3, D, OUT)
    M = jnp.einsum("jcf,tfn->tjcn", sel, w3)
    b2d = jnp.broadcast_to(b, (8, OUT))
    out = _project(p1, p2, p3, M, b2d)
    return out.reshape(B, OUT)
